# baseline (speedup 1.0000x reference)
import jax
import jax.numpy as jnp
from jax import lax
from jax.experimental import pallas as pl
from jax.experimental.pallas import tpu as pltpu
from jax.experimental.pallas import tpu_sc as plsc

_INFO = plsc.get_sparse_core_info()
_NC = _INFO.num_cores
_NS = _INFO.num_subcores
_NW = _NC * _NS

_GRP = 128
_FIRE = 8


def _gather_body(table_hbm, idx_hbm, out_hbm, idx_v, rows_v, sem):
    wid = lax.axis_index("s") * _NC + lax.axis_index("c")
    G = idx_hbm.shape[1]
    rows_per_outer = _FIRE * _GRP
    pltpu.sync_copy(idx_hbm.at[wid], idx_v)

    def outer(o, carry):
        handles = []
        for k in range(_FIRE):
            g = o * _FIRE + k
            handles.append(
                pltpu.async_copy(
                    table_hbm.at[idx_v.at[g]],
                    rows_v.at[pl.ds(k * _GRP, _GRP)],
                    sem,
                )
            )
        for h in handles:
            h.wait()
        off = pl.multiple_of(
            wid * (G * _GRP) + o * rows_per_outer, rows_per_outer
        )
        pltpu.sync_copy(rows_v, out_hbm.at[pl.ds(off, rows_per_outer)])
        return carry

    lax.fori_loop(0, G // _FIRE, outer, 0)


def _sc_gather(table, idx3, n_rows):
    D = table.shape[1]
    G = idx3.shape[1]
    call = pl.kernel(
        _gather_body,
        out_type=jax.ShapeDtypeStruct((n_rows, D), jnp.float32),
        mesh=plsc.VectorSubcoreMesh(core_axis_name="c", subcore_axis_name="s"),
        scratch_types=[
            pltpu.VMEM((G, _GRP), jnp.int32),
            pltpu.VMEM((_FIRE * _GRP, D), jnp.float32),
            pltpu.SemaphoreType.DMA,
        ],
        compiler_params=pltpu.CompilerParams(use_tc_tiling_on_sc=False),
    )
    return call(table, idx3)


_BB = 4096


_TCH = 512


def _trans_chunk(in_v, out_v, D):
    W = in_v.shape[1] - 1
    rows0 = lax.iota(jnp.int32, 16)
    rows1 = rows0 + 16

    def inner(u, carry2):
        for s in range(4):
            cols = jnp.full((16,), u * 4 + s, jnp.int32)
            for h, rows in ((0, rows0), (1, rows1)):
                vec = plsc.load_gather(in_v, [rows, cols])
                out_v[u, pl.ds(32 * s + 16 * h, 16)] = vec
        return carry2

    lax.fori_loop(0, W // 4, inner, 0)


def _transpose_body(tT_hbm, tail_hbm, out_hbm, in0, in1, out0, out1, outt,
                    s0, s1, w0, w1):
    wid = lax.axis_index("s") * _NC + lax.axis_index("c")
    D = tT_hbm.shape[0]
    V = tT_hbm.shape[1]
    nfull = V // _TCH
    tail = V - nfull * _TCH
    orows = _TCH * D // 128
    n_mine = (nfull - wid + _NW - 1) // _NW
    ins = (in0, in1)
    outs = (out0, out1)
    ssems = (s0, s1)
    wsems = (w0, w1)

    def stage_start(i, t):
        base = pl.multiple_of((wid + i * _NW) * _TCH, _TCH)
        pltpu.async_copy(tT_hbm.at[:, pl.ds(base, _TCH)],
                         ins[t].at[:, pl.ds(0, _TCH)], ssems[t])

    def stage_wait(t):
        pltpu.make_async_copy(
            tT_hbm.at[:, pl.ds(0, _TCH)],
            ins[t].at[:, pl.ds(0, _TCH)], ssems[t]).wait()

    def put_start(i, t):
        base = pl.multiple_of((wid + i * _NW) * orows, 8)
        pltpu.async_copy(outs[t], out_hbm.at[pl.ds(base, orows)], wsems[t])

    def put_wait(t):
        pltpu.make_async_copy(
            outs[t], out_hbm.at[pl.ds(0, orows)], wsems[t]).wait()

    @pl.when(n_mine > 0)
    def _prologue():
        stage_start(0, 0)

    def body(k, carry):
        for t in range(2):
            i = 2 * k + t

            @pl.when(i < n_mine)
            def _work():
                stage_wait(t)

                @pl.when(i + 1 < n_mine)
                def _pref():
                    stage_start(i + 1, 1 - t)

                @pl.when(i >= 2)
                def _drain():
                    put_wait(t)

                _trans_chunk(ins[t], outs[t], D)
                put_start(i, t)
        return carry

    lax.fori_loop(0, (n_mine + 1) // 2, body, 0)

    @pl.when(n_mine >= 1)
    def _drain0():
        put_wait(0)

    @pl.when(n_mine >= 2)
    def _drain1():
        put_wait(1)

    @pl.when(wid == 0)
    def _tail():
        pltpu.sync_copy(tail_hbm, outt)
        pltpu.sync_copy(
            outt, out_hbm.at[pl.ds(pl.multiple_of(nfull * orows, 8),
                                   tail * D // 128)])


def _sc_transpose(tT, tail16):
    D, V = tT.shape
    tail = V % _TCH
    call = pl.kernel(
        _transpose_body,
        out_type=jax.ShapeDtypeStruct((V // 4, 128), jnp.float32),
        mesh=plsc.VectorSubcoreMesh(core_axis_name="c", subcore_axis_name="s"),
        scratch_types=[
            pltpu.VMEM((D, _TCH + 1), jnp.float32),
            pltpu.VMEM((D, _TCH + 1), jnp.float32),
            pltpu.VMEM((_TCH * D // 128, 128), jnp.float32),
            pltpu.VMEM((_TCH * D // 128, 128), jnp.float32),
            pltpu.VMEM((tail * D // 128, 128), jnp.float32),
            pltpu.SemaphoreType.DMA,
            pltpu.SemaphoreType.DMA,
            pltpu.SemaphoreType.DMA,
            pltpu.SemaphoreType.DMA,
        ],
        compiler_params=pltpu.CompilerParams(
            use_tc_tiling_on_sc=True, needs_layout_passes=False),
    )
    return call(tT, tail16)


def _proj_body(e_ref, wt_ref, b_ref, out_ref):
    wt = wt_ref[...]
    bcol = b_ref[...]
    e = e_ref[...]
    parts = [
        jax.lax.dot_general(
            wt,
            e[:, 32 * q : 32 * (q + 1)],
            (((1,), (1,)), ((), ())),
            preferred_element_type=jnp.float32,
        )
        for q in range(4)
    ]
    yt = jnp.concatenate(parts, axis=1)
    out_ref[...] = (yt + bcol)[None]


def _tc_project(emb2, Wt, b2, B, L):
    Dout, D = Wt.shape
    nI = B // _BB
    M = _BB // 4
    return pl.pallas_call(
        _proj_body,
        grid=(L, nI),
        in_specs=[
            pl.BlockSpec((M, 128), lambda l, i: (l * nI + i, 0)),
            pl.BlockSpec((Dout, D), lambda l, i: (0, 0)),
            pl.BlockSpec((Dout, 1), lambda l, i: (0, 0)),
        ],
        out_specs=pl.BlockSpec((1, Dout, _BB), lambda l, i: (l, 0, i)),
        out_shape=jax.ShapeDtypeStruct((L, Dout, B), jnp.float32),
    )(emb2, Wt, b2)


def kernel(x, table, W, b):
    B, L = x.shape
    Dout = W.shape[1]
    N = B * L
    Q = _BB // 4
    xT = jnp.transpose(x).astype(jnp.int32)
    idxp = xT.reshape(N // _BB, 4, Q).transpose(0, 2, 1).reshape(N)
    idx3 = idxp.reshape(_NW, N // (_NW * _GRP), _GRP)
    V = table.shape[0]
    D = table.shape[1]
    nt = V % _TCH
    tail16 = jax.lax.slice(table, (V - nt, 0), (V, D)).reshape(nt * D // 128, 128)
    table_rm = _sc_transpose(jnp.transpose(table), tail16).reshape(V, D)
    emb = _sc_gather(table_rm, idx3, N)
    emb2 = emb.reshape(N // 4, 128)
    outT = _tc_project(emb2, jnp.transpose(W), b.reshape(Dout, 1), B, L)
    return jnp.transpose(outT, (2, 0, 1))


# speedup vs baseline: 1.1537x; 1.1537x over previous
import jax
import jax.numpy as jnp
from jax import lax
from jax.experimental import pallas as pl
from jax.experimental.pallas import tpu as pltpu
from jax.experimental.pallas import tpu_sc as plsc

_INFO = plsc.get_sparse_core_info()
_NC = _INFO.num_cores
_NS = _INFO.num_subcores
_NW = _NC * _NS

_GRP = 128
_FIRE = 8


def _gather_body(table_hbm, idx_hbm, out_hbm, idx_v, rows_v, sem):
    wid = lax.axis_index("s") * _NC + lax.axis_index("c")
    G = idx_hbm.shape[1]
    rows_per_outer = _FIRE * _GRP
    pltpu.sync_copy(idx_hbm.at[wid], idx_v)

    def outer(o, carry):
        handles = []
        for k in range(_FIRE):
            g = o * _FIRE + k
            handles.append(
                pltpu.async_copy(
                    table_hbm.at[idx_v.at[g]],
                    rows_v.at[pl.ds(k * _GRP, _GRP)],
                    sem,
                )
            )
        for h in handles:
            h.wait()
        off = pl.multiple_of(
            wid * (G * _GRP) + o * rows_per_outer, rows_per_outer
        )
        pltpu.sync_copy(rows_v, out_hbm.at[pl.ds(off, rows_per_outer)])
        return carry

    lax.fori_loop(0, G // _FIRE, outer, 0)


def _sc_gather(table, idx3, n_rows):
    D = table.shape[1]
    G = idx3.shape[1]
    call = pl.kernel(
        _gather_body,
        out_type=jax.ShapeDtypeStruct((n_rows, D), jnp.float32),
        mesh=plsc.VectorSubcoreMesh(core_axis_name="c", subcore_axis_name="s"),
        scratch_types=[
            pltpu.VMEM((G, _GRP), jnp.int32),
            pltpu.VMEM((_FIRE * _GRP, D), jnp.float32),
            pltpu.SemaphoreType.DMA,
        ],
        compiler_params=pltpu.CompilerParams(use_tc_tiling_on_sc=False),
    )
    return call(table, idx3)


_BB = 4096


_TCH = 512


def _trans_chunk(in_v, out_v, D):
    W = in_v.shape[1]
    lane = lax.iota(jnp.int32, 16)

    def inner(j, carry2):
        rbase = j * (16 * D) + lane * D
        for d in range(D):
            flat = rbase + d
            plsc.store_scatter(
                out_v,
                [lax.shift_right_logical(flat, 7), lax.bitwise_and(flat, 127)],
                in_v[d, pl.ds(j * 16, 16)],
            )
        return carry2

    lax.fori_loop(0, W // 16, inner, 0)


def _transpose_body(tT_hbm, tail_hbm, out_hbm, in0, in1, out0, out1, outt,
                    s0, s1, w0, w1):
    wid = lax.axis_index("s") * _NC + lax.axis_index("c")
    D = tT_hbm.shape[0]
    V = tT_hbm.shape[1]
    nfull = V // _TCH
    tail = V - nfull * _TCH
    orows = _TCH * D // 128
    n_mine = (nfull - wid + _NW - 1) // _NW
    ins = (in0, in1)
    outs = (out0, out1)
    ssems = (s0, s1)
    wsems = (w0, w1)

    def stage_start(i, t):
        base = pl.multiple_of((wid + i * _NW) * _TCH, _TCH)
        pltpu.async_copy(tT_hbm.at[:, pl.ds(base, _TCH)], ins[t], ssems[t])

    def stage_wait(t):
        pltpu.make_async_copy(
            tT_hbm.at[:, pl.ds(0, _TCH)], ins[t], ssems[t]).wait()

    def put_start(i, t):
        base = pl.multiple_of((wid + i * _NW) * orows, 8)
        pltpu.async_copy(outs[t].at[:, pl.ds(0, 128)],
                         out_hbm.at[pl.ds(base, orows)], wsems[t])

    def put_wait(t):
        pltpu.make_async_copy(
            outs[t].at[:, pl.ds(0, 128)],
            out_hbm.at[pl.ds(0, orows)], wsems[t]).wait()

    @pl.when(n_mine > 0)
    def _prologue():
        stage_start(0, 0)

    def body(k, carry):
        for t in range(2):
            i = 2 * k + t

            @pl.when(i < n_mine)
            def _work():
                stage_wait(t)

                @pl.when(i + 1 < n_mine)
                def _pref():
                    stage_start(i + 1, 1 - t)

                @pl.when(i >= 2)
                def _drain():
                    put_wait(t)

                _trans_chunk(ins[t], outs[t], D)
                put_start(i, t)
        return carry

    lax.fori_loop(0, (n_mine + 1) // 2, body, 0)

    @pl.when(n_mine >= 1)
    def _drain0():
        put_wait(0)

    @pl.when(n_mine >= 2)
    def _drain1():
        put_wait(1)

    @pl.when(wid == 0)
    def _tail():
        pltpu.sync_copy(tail_hbm, outt)
        pltpu.sync_copy(
            outt, out_hbm.at[pl.ds(pl.multiple_of(nfull * orows, 8),
                                   tail * D // 128)])


def _sc_transpose(tT, tail16):
    D, V = tT.shape
    tail = V % _TCH
    call = pl.kernel(
        _transpose_body,
        out_type=jax.ShapeDtypeStruct((V // 4, 128), jnp.float32),
        mesh=plsc.VectorSubcoreMesh(core_axis_name="c", subcore_axis_name="s"),
        scratch_types=[
            pltpu.VMEM((D, _TCH), jnp.float32),
            pltpu.VMEM((D, _TCH), jnp.float32),
            pltpu.VMEM((_TCH * D // 128, 132), jnp.float32),
            pltpu.VMEM((_TCH * D // 128, 132), jnp.float32),
            pltpu.VMEM((tail * D // 128, 128), jnp.float32),
            pltpu.SemaphoreType.DMA,
            pltpu.SemaphoreType.DMA,
            pltpu.SemaphoreType.DMA,
            pltpu.SemaphoreType.DMA,
        ],
        compiler_params=pltpu.CompilerParams(
            use_tc_tiling_on_sc=True, needs_layout_passes=False),
    )
    return call(tT, tail16)


def _proj_body(e_ref, wt_ref, b_ref, out_ref):
    wt = wt_ref[...]
    bcol = b_ref[...]
    e = e_ref[...]
    parts = [
        jax.lax.dot_general(
            wt,
            e[:, 32 * q : 32 * (q + 1)],
            (((1,), (1,)), ((), ())),
            preferred_element_type=jnp.float32,
        )
        for q in range(4)
    ]
    yt = jnp.concatenate(parts, axis=1)
    out_ref[...] = (yt + bcol)[None]


def _tc_project(emb2, Wt, b2, B, L):
    Dout, D = Wt.shape
    nI = B // _BB
    M = _BB // 4
    return pl.pallas_call(
        _proj_body,
        grid=(L, nI),
        in_specs=[
            pl.BlockSpec((M, 128), lambda l, i: (l * nI + i, 0)),
            pl.BlockSpec((Dout, D), lambda l, i: (0, 0)),
            pl.BlockSpec((Dout, 1), lambda l, i: (0, 0)),
        ],
        out_specs=pl.BlockSpec((1, Dout, _BB), lambda l, i: (l, 0, i)),
        out_shape=jax.ShapeDtypeStruct((L, Dout, B), jnp.float32),
    )(emb2, Wt, b2)


def kernel(x, table, W, b):
    B, L = x.shape
    Dout = W.shape[1]
    N = B * L
    Q = _BB // 4
    xT = jnp.transpose(x).astype(jnp.int32)
    idxp = xT.reshape(N // _BB, 4, Q).transpose(0, 2, 1).reshape(N)
    idx3 = idxp.reshape(_NW, N // (_NW * _GRP), _GRP)
    V = table.shape[0]
    D = table.shape[1]
    nt = V % _TCH
    tail16 = jax.lax.slice(table, (V - nt, 0), (V, D)).reshape(nt * D // 128, 128)
    table_rm = _sc_transpose(jnp.transpose(table), tail16).reshape(V, D)
    emb = _sc_gather(table_rm, idx3, N)
    emb2 = emb.reshape(N // 4, 128)
    outT = _tc_project(emb2, jnp.transpose(W), b.reshape(Dout, 1), B, L)
    return jnp.transpose(outT, (2, 0, 1))


# speedup vs baseline: 1.3008x; 1.1274x over previous
import jax
import jax.numpy as jnp
from jax import lax
from jax.experimental import pallas as pl
from jax.experimental.pallas import tpu as pltpu
from jax.experimental.pallas import tpu_sc as plsc

_INFO = plsc.get_sparse_core_info()
_NC = _INFO.num_cores
_NS = _INFO.num_subcores
_NW = _NC * _NS

_GRP = 128
_FIRE = 8


def _gather_body(table_hbm, idx_hbm, out_hbm, idx_v, rows_v, sem):
    wid = lax.axis_index("s") * _NC + lax.axis_index("c")
    G = idx_hbm.shape[1]
    rows_per_outer = _FIRE * _GRP
    pltpu.sync_copy(idx_hbm.at[wid], idx_v)

    def outer(o, carry):
        handles = []
        for k in range(_FIRE):
            g = o * _FIRE + k
            handles.append(
                pltpu.async_copy(
                    table_hbm.at[idx_v.at[g]],
                    rows_v.at[pl.ds(k * _GRP, _GRP)],
                    sem,
                )
            )
        for h in handles:
            h.wait()
        off = pl.multiple_of(
            wid * (G * _GRP) + o * rows_per_outer, rows_per_outer
        )
        pltpu.sync_copy(rows_v, out_hbm.at[pl.ds(off, rows_per_outer)])
        return carry

    lax.fori_loop(0, G // _FIRE, outer, 0)


def _sc_gather(table, idx3, n_rows):
    D = table.shape[1]
    G = idx3.shape[1]
    call = pl.kernel(
        _gather_body,
        out_type=jax.ShapeDtypeStruct((n_rows, D), jnp.float32),
        mesh=plsc.VectorSubcoreMesh(core_axis_name="c", subcore_axis_name="s"),
        scratch_types=[
            pltpu.VMEM((G, _GRP), jnp.int32),
            pltpu.VMEM((_FIRE * _GRP, D), jnp.float32),
            pltpu.SemaphoreType.DMA,
        ],
        compiler_params=pltpu.CompilerParams(use_tc_tiling_on_sc=False),
    )
    return call(table, idx3)


_BB = 4096


def _proj_body(e_ref, wt_ref, b_ref, out_ref):
    wt = wt_ref[...]
    bcol = b_ref[...]
    e = e_ref[...]
    parts = [
        jax.lax.dot_general(
            wt,
            e[:, 32 * q : 32 * (q + 1)],
            (((1,), (1,)), ((), ())),
            preferred_element_type=jnp.float32,
        )
        for q in range(4)
    ]
    yt = jnp.concatenate(parts, axis=1)
    out_ref[...] = (yt + bcol)[None]


def _tc_project(emb2, Wt, b2, B, L):
    Dout, D = Wt.shape
    nI = B // _BB
    M = _BB // 4
    return pl.pallas_call(
        _proj_body,
        grid=(L, nI),
        in_specs=[
            pl.BlockSpec((M, 128), lambda l, i: (l * nI + i, 0)),
            pl.BlockSpec((Dout, D), lambda l, i: (0, 0)),
            pl.BlockSpec((Dout, 1), lambda l, i: (0, 0)),
        ],
        out_specs=pl.BlockSpec((1, Dout, _BB), lambda l, i: (l, 0, i)),
        out_shape=jax.ShapeDtypeStruct((L, Dout, B), jnp.float32),
    )(emb2, Wt, b2)


def kernel(x, table, W, b):
    B, L = x.shape
    Dout = W.shape[1]
    N = B * L
    Q = _BB // 4
    xT = jnp.transpose(x).astype(jnp.int32)
    idxp = xT.reshape(N // _BB, 4, Q).transpose(0, 2, 1).reshape(N)
    idx3 = idxp.reshape(_NW, N // (_NW * _GRP), _GRP)
    emb = _sc_gather(table, idx3, N)
    emb2 = emb.reshape(N // 4, 128)
    outT = _tc_project(emb2, jnp.transpose(W), b.reshape(Dout, 1), B, L)
    return jnp.transpose(outT, (2, 0, 1))
